# Initial kernel scaffold; baseline (speedup 1.0000x reference)
import jax
import jax.numpy as jnp
from jax.experimental import pallas as pl


def kernel(x, location, bias):
    raise NotImplementedError("write your pallas kernel here")


# speedup vs baseline: 6.5632x; 6.5632x over previous
import functools

import jax
import jax.numpy as jnp
from jax import lax
from jax.experimental import pallas as pl
from jax.experimental.pallas import tpu as pltpu
from jax.experimental.pallas import tpu_sc as plsc

B = 4
C = 288
P = 9
H = 128
W = 128
HW = H * W
K = 4
NJOBS = (B * C) // K
NWORKERS = 32
JOBS_PER_W = NJOBS // NWORKERS
CHUNK = 2048
NCHUNK = HW // CHUNK
VECS = CHUNK // 16


def _sc_body(x_hbm, loc_hbm, bias_hbm, out_hbm,
             planes_v, lx_v, ly_v, out_v, bias_v):
    cid = lax.axis_index("c")
    sid = lax.axis_index("s")
    wid = sid * 2 + cid

    pltpu.sync_copy(bias_hbm, bias_v)

    def job_body(j, _):
        job = wid * JOBS_PER_W + j
        cb = job % 8
        grp = job // 8
        b = grp // P
        p = grp % P
        chan0 = p * 32 + cb * K
        row0 = b * C + chan0

        pltpu.sync_copy(x_hbm.at[pl.ds(row0 * HW, K * HW)], planes_v)

        lrow_x = (b * 2 * P + 2 * p) * HW
        lrow_y = lrow_x + HW

        biases = tuple(
            plsc.load_gather(
                bias_v,
                [jnp.broadcast_to(chan0 + k, (16,)).astype(jnp.int32)])
            for k in range(K))

        def chunk_body(s, _):
            off = s * CHUNK
            pltpu.sync_copy(loc_hbm.at[pl.ds(lrow_x + off, CHUNK)], lx_v)
            pltpu.sync_copy(loc_hbm.at[pl.ds(lrow_y + off, CHUNK)], ly_v)

            def vec_body(v, _):
                lx = lx_v[pl.ds(v * 16, 16)]
                ly = ly_v[pl.ds(v * 16, 16)]
                x0i = lx.astype(jnp.int32)
                x0f = x0i.astype(jnp.float32)
                xneg = x0f > lx
                x0i = jnp.where(xneg, x0i - 1, x0i)
                x0f = jnp.where(xneg, x0f - 1.0, x0f)
                y0i = ly.astype(jnp.int32)
                y0f = y0i.astype(jnp.float32)
                yneg = y0f > ly
                y0i = jnp.where(yneg, y0i - 1, y0i)
                y0f = jnp.where(yneg, y0f - 1.0, y0f)

                fx = lx - x0f
                fy = ly - y0f
                x0c = jnp.minimum(jnp.maximum(x0i, 0), W - 1)
                x1c = jnp.minimum(jnp.maximum(x0i + 1, 0), W - 1)
                y0c = jnp.minimum(jnp.maximum(y0i, 0), H - 1)
                y1c = jnp.minimum(jnp.maximum(y0i + 1, 0), H - 1)

                r0 = y0c * W
                r1 = y1c * W
                i00 = r0 + x0c
                i01 = r0 + x1c
                i10 = r1 + x0c
                i11 = r1 + x1c

                wx0 = 1.0 - fx
                wy0 = 1.0 - fy
                w00 = wy0 * wx0
                w01 = wy0 * fx
                w10 = fy * wx0
                w11 = fy * fx

                for k in range(K):
                    o = k * HW
                    v00 = plsc.load_gather(planes_v, [i00 + o])
                    v01 = plsc.load_gather(planes_v, [i01 + o])
                    v10 = plsc.load_gather(planes_v, [i10 + o])
                    v11 = plsc.load_gather(planes_v, [i11 + o])
                    acc = (w00 * v00 + w01 * v01 + w10 * v10 + w11 * v11
                           + biases[k])
                    out_v[pl.ds(k * CHUNK + v * 16, 16)] = acc
                return 0

            lax.fori_loop(0, VECS, vec_body, 0)

            for k in range(K):
                pltpu.sync_copy(
                    out_v.at[pl.ds(k * CHUNK, CHUNK)],
                    out_hbm.at[pl.ds((row0 + k) * HW + off, CHUNK)])
            return 0

        lax.fori_loop(0, NCHUNK, chunk_body, 0)
        return 0

    lax.fori_loop(0, JOBS_PER_W, job_body, 0)


_sc_collect = functools.partial(
    pl.kernel,
    out_type=jax.ShapeDtypeStruct((B * C * HW,), jnp.float32),
    mesh=plsc.VectorSubcoreMesh(core_axis_name="c", subcore_axis_name="s"),
    compiler_params=pltpu.CompilerParams(needs_layout_passes=False),
    scratch_types=[
        pltpu.VMEM((K * HW,), jnp.float32),
        pltpu.VMEM((CHUNK,), jnp.float32),
        pltpu.VMEM((CHUNK,), jnp.float32),
        pltpu.VMEM((K * CHUNK,), jnp.float32),
        pltpu.VMEM((C,), jnp.float32),
    ],
)(_sc_body)


@jax.jit
def kernel(x, location, bias):
    out = _sc_collect(x.reshape(-1), location.reshape(-1), bias)
    return out.reshape(B, C, H, W)


# speedup vs baseline: 8.6219x; 1.3137x over previous
import functools

import jax
import jax.numpy as jnp
from jax import lax
from jax.experimental import pallas as pl
from jax.experimental.pallas import tpu as pltpu
from jax.experimental.pallas import tpu_sc as plsc

B = 4
C = 288
P = 9
H = 128
W = 128
HW = H * W
K = 4
NJOBS = (B * C) // K
NWORKERS = 32
JOBS_PER_W = NJOBS // NWORKERS
CHUNK = 2048
NCHUNK = HW // CHUNK
VECS = CHUNK // 16


def _job_params(job):
    cb = job % 8
    grp = job // 8
    b = grp // P
    p = grp % P
    chan0 = p * 32 + cb * K
    row0 = b * C + chan0
    lrow = (b * 2 * P + 2 * p) * HW
    return chan0, row0, lrow


def _sc_body(x_hbm, loc_hbm, bias_hbm, out_hbm,
             planes_v, loc_v, out_v, bias_v,
             plane_sem, loc_sem, out_sem):
    cid = lax.axis_index("c")
    sid = lax.axis_index("s")
    wid = sid * 2 + cid

    pltpu.sync_copy(bias_hbm, bias_v)

    _, row0_0, lrow_0 = _job_params(wid * JOBS_PER_W)
    pltpu.async_copy(x_hbm.at[pl.ds(row0_0 * HW, K * HW)], planes_v,
                     plane_sem)
    pltpu.async_copy(loc_hbm.at[pl.ds(lrow_0, 2 * HW)], loc_v, loc_sem)

    def job_body(j, _):
        job = wid * JOBS_PER_W + j
        chan0, row0, lrow = _job_params(job)

        pltpu.make_async_copy(
            x_hbm.at[pl.ds(row0 * HW, K * HW)], planes_v, plane_sem).wait()
        pltpu.make_async_copy(
            loc_hbm.at[pl.ds(lrow, 2 * HW)], loc_v, loc_sem).wait()

        biases = tuple(
            plsc.load_gather(
                bias_v,
                [jnp.broadcast_to(chan0 + k, (16,)).astype(jnp.int32)])
            for k in range(K))

        def make_vec_body(s, buf):
            def vec_body(v, _):
                lx = loc_v[pl.ds(s * CHUNK + v * 16, 16)]
                ly = loc_v[pl.ds(HW + s * CHUNK + v * 16, 16)]
                x0i = lx.astype(jnp.int32)
                x0f = x0i.astype(jnp.float32)
                xneg = x0f > lx
                x0i = jnp.where(xneg, x0i - 1, x0i)
                x0f = jnp.where(xneg, x0f - 1.0, x0f)
                y0i = ly.astype(jnp.int32)
                y0f = y0i.astype(jnp.float32)
                yneg = y0f > ly
                y0i = jnp.where(yneg, y0i - 1, y0i)
                y0f = jnp.where(yneg, y0f - 1.0, y0f)

                fx = lx - x0f
                fy = ly - y0f
                x0c = jnp.minimum(jnp.maximum(x0i, 0), W - 1)
                x1c = jnp.minimum(jnp.maximum(x0i + 1, 0), W - 1)
                y0c = jnp.minimum(jnp.maximum(y0i, 0), H - 1)
                y1c = jnp.minimum(jnp.maximum(y0i + 1, 0), H - 1)

                r0 = y0c * W
                r1 = y1c * W
                i00 = r0 + x0c
                i01 = r0 + x1c
                i10 = r1 + x0c
                i11 = r1 + x1c

                wx0 = 1.0 - fx
                wy0 = 1.0 - fy
                w00 = wy0 * wx0
                w01 = wy0 * fx
                w10 = fy * wx0
                w11 = fy * fx

                for k in range(K):
                    o = k * HW
                    v00 = plsc.load_gather(planes_v, [i00 + o])
                    v01 = plsc.load_gather(planes_v, [i01 + o])
                    v10 = plsc.load_gather(planes_v, [i10 + o])
                    v11 = plsc.load_gather(planes_v, [i11 + o])
                    acc = (w00 * v00 + w01 * v01 + w10 * v10 + w11 * v11
                           + biases[k])
                    out_v[pl.ds(buf * K * CHUNK + k * CHUNK + v * 16, 16)] \
                        = acc
                return 0
            return vec_body

        outcps = {}
        for s in range(NCHUNK):
            buf = s % 2
            if s >= 2:
                for d in outcps[s - 2]:
                    d.wait()
            lax.fori_loop(0, VECS, make_vec_body(s, buf), 0)
            if s == NCHUNK - 1:
                @pl.when(j < JOBS_PER_W - 1)
                def _():
                    _, nrow0, nlrow = _job_params(job + 1)
                    pltpu.async_copy(
                        x_hbm.at[pl.ds(nrow0 * HW, K * HW)], planes_v,
                        plane_sem)
                    pltpu.async_copy(
                        loc_hbm.at[pl.ds(nlrow, 2 * HW)], loc_v, loc_sem)
            outcps[s] = [
                pltpu.async_copy(
                    out_v.at[pl.ds(buf * K * CHUNK + k * CHUNK, CHUNK)],
                    out_hbm.at[pl.ds((row0 + k) * HW + s * CHUNK, CHUNK)],
                    out_sem)
                for k in range(K)]
        for s in (NCHUNK - 2, NCHUNK - 1):
            for d in outcps[s]:
                d.wait()
        return 0

    lax.fori_loop(0, JOBS_PER_W, job_body, 0)


_sc_collect = functools.partial(
    pl.kernel,
    out_type=jax.ShapeDtypeStruct((B * C * HW,), jnp.float32),
    mesh=plsc.VectorSubcoreMesh(core_axis_name="c", subcore_axis_name="s"),
    compiler_params=pltpu.CompilerParams(needs_layout_passes=False),
    scratch_types=[
        pltpu.VMEM((K * HW,), jnp.float32),
        pltpu.VMEM((2 * HW,), jnp.float32),
        pltpu.VMEM((2 * K * CHUNK,), jnp.float32),
        pltpu.VMEM((C,), jnp.float32),
        pltpu.SemaphoreType.DMA,
        pltpu.SemaphoreType.DMA,
        pltpu.SemaphoreType.DMA,
    ],
)(_sc_body)


@jax.jit
def kernel(x, location, bias):
    out = _sc_collect(x.reshape(-1), location.reshape(-1), bias)
    return out.reshape(B, C, H, W)
